# baseline (speedup 1.0000x reference)
import functools

import jax
import jax.numpy as jnp
from jax import lax
from jax.experimental import pallas as pl
from jax.experimental.pallas import tpu as pltpu
from jax.experimental.pallas import tpu_sc as plsc

K = 64
LANES = 16
CAP = 128


def _encode_body(x_ref, w_ref, b_ref, pre_ref):
    acc = jax.lax.dot_general(
        x_ref[...], w_ref[...], (((1,), (1,)), ((), ())),
        preferred_element_type=jnp.float32)
    pre_ref[...] = acc + b_ref[...]


def _decode_body(lat_ref, wd_ref, out_ref):
    j = pl.program_id(0)
    i = pl.program_id(1)
    rb = lat_ref.shape[0]
    acc = jax.lax.dot_general(
        lat_ref[...].astype(jnp.bfloat16), wd_ref[...],
        (((1,), (1,)), ((), ())), preferred_element_type=jnp.float32)
    rows = pl.ds(i * rb, rb)

    @pl.when(j == 0)
    def _init():
        out_ref[rows, :] = acc

    @pl.when(j != 0)
    def _acc():
        out_ref[rows, :] = out_ref[rows, :] + acc


def _topk_sc_body(n_rows, d_sae, rows_per_w, pre_hbm, lat_hbm,
                  buf, outb, cand, candk, nvref):
    nc = 2
    wid = lax.axis_index("s") * nc + lax.axis_index("c")
    nv_row = d_sae // LANES
    lane = lax.iota(jnp.int32, LANES)
    ninf = jnp.full((LANES,), -jnp.inf, jnp.float32)
    int_min = jnp.full((LANES,), jnp.int32(-2147483647 - 1))

    def _allreduce(x, op):
        for s in (1, 2, 4, 8):
            x = op(x, x.at[lane ^ s].get(mode="promise_in_bounds"))
        return x

    def row_body(r, _):
        row = wid * rows_per_w + r
        pltpu.sync_copy(pre_hbm.at[row], buf)

        def p1(i, ms):
            b = i * (4 * LANES)
            m0 = jnp.maximum(ms[0], buf[pl.ds(b, LANES)])
            m1 = jnp.maximum(ms[1], buf[pl.ds(b + LANES, LANES)])
            m2 = jnp.maximum(ms[2], buf[pl.ds(b + 2 * LANES, LANES)])
            m3 = jnp.maximum(ms[3], buf[pl.ds(b + 3 * LANES, LANES)])
            return (m0, m1, m2, m3)

        ms = lax.fori_loop(0, nv_row // 4, p1, (ninf, ninf, ninf, ninf))
        m = _allreduce(jnp.minimum(jnp.minimum(ms[0], ms[1]),
                                   jnp.minimum(ms[2], ms[3])),
                       jnp.minimum)

        def pz(i, _):
            cand[pl.ds(i * LANES, LANES)] = ninf
            return 0

        lax.fori_loop(0, CAP, pz, 0)

        def p2(i, cnt):
            v = buf[pl.ds(i * LANES, LANES)]
            ok = (v >= m) & (cnt < CAP)
            idx = cnt * LANES + lane
            plsc.store_scatter(cand, [idx], v, mask=ok)
            return cnt + jnp.where(ok, 1, 0)

        cnt = lax.fori_loop(0, nv_row, p2, jnp.zeros((LANES,), jnp.int32))
        nv = _allreduce(cnt, jnp.maximum)[0]

        def pk(i, _):
            ki = lax.bitcast_convert_type(cand[pl.ds(i * LANES, LANES)],
                                          jnp.int32)
            candk[pl.ds(i * LANES, LANES)] = jnp.where(
                ki < 0, ki ^ 0x7FFFFFFF, ki)
            return 0

        lax.fori_loop(0, nv, pk, 0)

        def sb(t, base):
            step = lax.shift_left(jnp.int32(1), jnp.int32(31) - t)
            candidate = base + step

            def cvec(i, acc):
                k = candk[pl.ds(i * LANES, LANES)]
                return acc + jnp.where(k >= candidate, 1, 0)

            accv = lax.fori_loop(0, nv, cvec,
                                 jnp.zeros((LANES,), jnp.int32))
            total = _allreduce(accv, jnp.add)
            return jnp.where(total >= K, candidate, base)

        base = lax.fori_loop(0, 32, sb, int_min)
        t64 = lax.bitcast_convert_type(
            jnp.where(base < 0, base ^ 0x7FFFFFFF, base), jnp.float32)

        def p4(i, _):
            b = i * (4 * LANES)
            for u in range(4):
                v = buf[pl.ds(b + u * LANES, LANES)]
                outb[pl.ds(b + u * LANES, LANES)] = jnp.where(
                    v >= t64, v, 0.0)
            return 0

        lax.fori_loop(0, nv_row // 4, p4, 0)
        pltpu.sync_copy(outb, lat_hbm.at[row])
        return 0

    lax.fori_loop(0, rows_per_w, row_body, 0)


def kernel(x, W_enc, b_enc, W_dec):
    n, d_model = x.shape
    d_sae = W_enc.shape[0]

    rb_e = min(1024, n)
    cb_e = min(2048, d_sae)
    b2 = b_enc.reshape(1, d_sae)
    pre = pl.pallas_call(
        _encode_body,
        grid=(n // rb_e, d_sae // cb_e),
        in_specs=[
            pl.BlockSpec((rb_e, d_model), lambda i, j: (i, 0)),
            pl.BlockSpec((cb_e, d_model), lambda i, j: (j, 0)),
            pl.BlockSpec((1, cb_e), lambda i, j: (0, j)),
        ],
        out_specs=pl.BlockSpec((rb_e, cb_e), lambda i, j: (i, j)),
        out_shape=jax.ShapeDtypeStruct((n, d_sae), jnp.float32),
    )(x, W_enc, b2)

    n_workers = 32
    rows_per_w = n // n_workers
    mesh = plsc.VectorSubcoreMesh(core_axis_name="c", subcore_axis_name="s")
    topk = functools.partial(
        pl.kernel,
        mesh=mesh,
        compiler_params=pltpu.CompilerParams(needs_layout_passes=False),
        out_type=jax.ShapeDtypeStruct((n, d_sae), jnp.float32),
        scratch_types=[
            pltpu.VMEM((d_sae,), jnp.float32),
            pltpu.VMEM((d_sae,), jnp.float32),
            pltpu.VMEM((CAP * LANES,), jnp.float32),
            pltpu.VMEM((CAP * LANES,), jnp.int32),
            pltpu.VMEM((LANES,), jnp.int32),
        ],
    )(functools.partial(_topk_sc_body, n, d_sae, rows_per_w))
    lat = topk(pre)

    rb_d = min(1024, n)
    cb_d = min(2048, d_sae)
    wd_bf = W_dec.astype(jnp.bfloat16)
    out = pl.pallas_call(
        _decode_body,
        grid=(d_sae // cb_d, n // rb_d),
        in_specs=[
            pl.BlockSpec((rb_d, cb_d), lambda j, i: (i, j)),
            pl.BlockSpec((d_model, cb_d), lambda j, i: (0, j)),
        ],
        out_specs=pl.BlockSpec((n, d_model), lambda j, i: (0, 0)),
        out_shape=jax.ShapeDtypeStruct((n, d_model), jnp.float32),
    )(lat, wd_bf)

    return (out, lat)


# speedup vs baseline: 1.1764x; 1.1764x over previous
import functools

import jax
import jax.numpy as jnp
from jax import lax
from jax.experimental import pallas as pl
from jax.experimental.pallas import tpu as pltpu
from jax.experimental.pallas import tpu_sc as plsc

K = 64
LANES = 16
CAP = 128


def _encode_body(x_ref, w_ref, b_ref, pre_ref):
    acc = jax.lax.dot_general(
        x_ref[...], w_ref[...], (((1,), (1,)), ((), ())),
        preferred_element_type=jnp.float32)
    pre_ref[...] = acc + b_ref[...]


def _decode_body(lat_ref, wd_ref, out_ref):
    j = pl.program_id(0)
    i = pl.program_id(1)
    rb = lat_ref.shape[0]
    acc = jax.lax.dot_general(
        lat_ref[...].astype(jnp.bfloat16), wd_ref[...],
        (((1,), (1,)), ((), ())), preferred_element_type=jnp.float32)
    rows = pl.ds(i * rb, rb)

    @pl.when(j == 0)
    def _init():
        out_ref[rows, :] = acc

    @pl.when(j != 0)
    def _acc():
        out_ref[rows, :] = out_ref[rows, :] + acc


def _topk_sc_body(n_rows, d_sae, rows_per_w, pre_hbm, lat_hbm,
                  buf0, buf1, ob0, ob1, cand, candk,
                  si0, si1, so0, so1):
    nc = 2
    wid = lax.axis_index("s") * nc + lax.axis_index("c")
    base_row = wid * rows_per_w
    nv_row = d_sae // LANES
    qcap = CAP // 4
    qwords = qcap * LANES
    lane = lax.iota(jnp.int32, LANES)
    ninf = jnp.full((LANES,), -jnp.inf, jnp.float32)
    int_min = jnp.full((LANES,), jnp.int32(-2147483647 - 1))

    def _allreduce(x, op):
        for s in (1, 2, 4, 8):
            x = op(x, x.at[lane ^ s].get(mode="promise_in_bounds"))
        return x

    bufs, obs = (buf0, buf1), (ob0, ob1)
    sis, sos = (si0, si1), (so0, so1)

    pltpu.async_copy(pre_hbm.at[base_row], buf0, si0)
    pltpu.async_copy(pre_hbm.at[base_row + 1], buf1, si1)

    def pair_body(rr, _):
        for kslot in range(2):
            buf, ob = bufs[kslot], obs[kslot]
            si, so = sis[kslot], sos[kslot]
            r = rr * 2 + kslot
            row = base_row + r
            pltpu.make_async_copy(pre_hbm.at[row], buf, si).wait()

            def p1(i, ms):
                b = i * (4 * LANES)
                m0 = jnp.maximum(ms[0], buf[pl.ds(b, LANES)])
                m1 = jnp.maximum(ms[1], buf[pl.ds(b + LANES, LANES)])
                m2 = jnp.maximum(ms[2], buf[pl.ds(b + 2 * LANES, LANES)])
                m3 = jnp.maximum(ms[3], buf[pl.ds(b + 3 * LANES, LANES)])
                return (m0, m1, m2, m3)

            ms = lax.fori_loop(0, nv_row // 4, p1, (ninf,) * 4,
                               unroll=4)
            m = _allreduce(jnp.minimum(jnp.minimum(ms[0], ms[1]),
                                       jnp.minimum(ms[2], ms[3])),
                           jnp.minimum)

            def pz(i, _):
                b = i * (4 * LANES)
                for u in range(4):
                    cand[pl.ds(b + u * LANES, LANES)] = ninf
                return 0

            lax.fori_loop(0, CAP // 4, pz, 0)

            def p2(i, idxc):
                b = i * (4 * LANES)
                out = []
                for u in range(4):
                    v = buf[pl.ds(b + u * LANES, LANES)]
                    ok = v >= m
                    plsc.store_scatter(cand, [idxc[u]], v, mask=ok)
                    out.append(idxc[u] + jnp.where(ok, LANES, 0))
                return tuple(out)

            idx0 = tuple(jnp.int32(q * qwords) + lane for q in range(4))
            idxc = lax.fori_loop(0, nv_row // 4, p2, idx0)
            hmax = jnp.maximum(jnp.maximum(idxc[0] - 0 * qwords,
                                           idxc[1] - 1 * qwords),
                               jnp.maximum(idxc[2] - 2 * qwords,
                                           idxc[3] - 3 * qwords))
            nv = lax.shift_right_logical(_allreduce(hmax, jnp.maximum)[0],
                                         4)

            def pk(i, _):
                b = i * LANES
                for q in range(4):
                    ki = lax.bitcast_convert_type(
                        cand[pl.ds(q * qwords + b, LANES)], jnp.int32)
                    candk[pl.ds(q * qwords + b, LANES)] = jnp.where(
                        ki < 0, ki ^ 0x7FFFFFFF, ki)
                return 0

            lax.fori_loop(0, nv, pk, 0)

            def sb(t, base):
                step = lax.shift_left(jnp.int32(1), jnp.int32(31) - t)
                candidate = base + step

                def cvec(i, acc):
                    b = i * LANES
                    for q in range(4):
                        k = candk[pl.ds(q * qwords + b, LANES)]
                        acc = acc + jnp.where(k >= candidate, 1, 0)
                    return acc

                accv = lax.fori_loop(0, nv, cvec,
                                     jnp.zeros((LANES,), jnp.int32))
                total = _allreduce(accv, jnp.add)
                return jnp.where(total >= K, candidate, base)

            base = lax.fori_loop(0, 32, sb, int_min)
            t64 = lax.bitcast_convert_type(
                jnp.where(base < 0, base ^ 0x7FFFFFFF, base), jnp.float32)

            @pl.when(r >= 2)
            def _():
                pltpu.make_async_copy(ob, lat_hbm.at[row - 2], so).wait()

            def p4(i, _):
                b = i * (4 * LANES)
                for u in range(4):
                    v = buf[pl.ds(b + u * LANES, LANES)]
                    ob[pl.ds(b + u * LANES, LANES)] = jnp.where(
                        v >= t64, v, 0.0)
                return 0

            lax.fori_loop(0, nv_row // 4, p4, 0, unroll=2)
            pltpu.async_copy(ob, lat_hbm.at[row], so)

            @pl.when(r + 2 < rows_per_w)
            def _():
                pltpu.async_copy(pre_hbm.at[row + 2], buf, si)
        return 0

    lax.fori_loop(0, rows_per_w // 2, pair_body, 0)

    pltpu.make_async_copy(ob0, lat_hbm.at[base_row + rows_per_w - 2],
                          so0).wait()
    pltpu.make_async_copy(ob1, lat_hbm.at[base_row + rows_per_w - 1],
                          so1).wait()


def kernel(x, W_enc, b_enc, W_dec):
    n, d_model = x.shape
    d_sae = W_enc.shape[0]

    rb_e = min(1024, n)
    cb_e = min(2048, d_sae)
    b2 = b_enc.reshape(1, d_sae)
    pre = pl.pallas_call(
        _encode_body,
        grid=(n // rb_e, d_sae // cb_e),
        in_specs=[
            pl.BlockSpec((rb_e, d_model), lambda i, j: (i, 0)),
            pl.BlockSpec((cb_e, d_model), lambda i, j: (j, 0)),
            pl.BlockSpec((1, cb_e), lambda i, j: (0, j)),
        ],
        out_specs=pl.BlockSpec((rb_e, cb_e), lambda i, j: (i, j)),
        out_shape=jax.ShapeDtypeStruct((n, d_sae), jnp.float32),
    )(x, W_enc, b2)

    n_workers = 32
    rows_per_w = n // n_workers
    mesh = plsc.VectorSubcoreMesh(core_axis_name="c", subcore_axis_name="s")
    topk = functools.partial(
        pl.kernel,
        mesh=mesh,
        compiler_params=pltpu.CompilerParams(needs_layout_passes=False),
        out_type=jax.ShapeDtypeStruct((n, d_sae), jnp.float32),
        scratch_types=[
            pltpu.VMEM((d_sae,), jnp.float32),
            pltpu.VMEM((d_sae,), jnp.float32),
            pltpu.VMEM((d_sae,), jnp.float32),
            pltpu.VMEM((d_sae,), jnp.float32),
            pltpu.VMEM((CAP * LANES,), jnp.float32),
            pltpu.VMEM((CAP * LANES,), jnp.int32),
            pltpu.SemaphoreType.DMA,
            pltpu.SemaphoreType.DMA,
            pltpu.SemaphoreType.DMA,
            pltpu.SemaphoreType.DMA,
        ],
    )(functools.partial(_topk_sc_body, n, d_sae, rows_per_w))
    lat = topk(pre)

    rb_d = min(1024, n)
    cb_d = min(2048, d_sae)
    wd_bf = W_dec.astype(jnp.bfloat16)
    out = pl.pallas_call(
        _decode_body,
        grid=(d_sae // cb_d, n // rb_d),
        in_specs=[
            pl.BlockSpec((rb_d, cb_d), lambda j, i: (i, j)),
            pl.BlockSpec((d_model, cb_d), lambda j, i: (0, j)),
        ],
        out_specs=pl.BlockSpec((n, d_model), lambda j, i: (0, 0)),
        out_shape=jax.ShapeDtypeStruct((n, d_model), jnp.float32),
    )(lat, wd_bf)

    return (out, lat)


# speedup vs baseline: 3.2969x; 2.8024x over previous
import functools

import jax
import jax.numpy as jnp
from jax import lax
from jax.experimental import pallas as pl
from jax.experimental.pallas import tpu as pltpu
from jax.experimental.pallas import tpu_sc as plsc

K = 64
LANES = 16
CAP = 128


def _encode_body(x_ref, w_ref, b_ref, pre_ref):
    acc = jax.lax.dot_general(
        x_ref[...], w_ref[...], (((1,), (1,)), ((), ())),
        preferred_element_type=jnp.float32)
    pre_ref[...] = acc + b_ref[...]


def _decode_body(lat_ref, wd_ref, out_ref):
    j = pl.program_id(0)
    i = pl.program_id(1)
    rb = lat_ref.shape[0]
    acc = jax.lax.dot_general(
        lat_ref[...].astype(jnp.bfloat16), wd_ref[...],
        (((1,), (1,)), ((), ())), preferred_element_type=jnp.float32)
    rows = pl.ds(i * rb, rb)

    @pl.when(j == 0)
    def _init():
        out_ref[rows, :] = acc

    @pl.when(j != 0)
    def _acc():
        out_ref[rows, :] = out_ref[rows, :] + acc


def _topk_sc_body(n_rows, d_sae, rows_per_w, pre_hbm, lat_hbm,
                  buf0, buf1, ob0, ob1, cand, candk,
                  si0, si1, so0, so1):
    nc = 2
    wid = lax.axis_index("s") * nc + lax.axis_index("c")
    base_row = wid * rows_per_w
    nv_row = d_sae // LANES
    qcap = CAP // 4
    qwords = qcap * LANES
    lane = lax.iota(jnp.int32, LANES)
    ninf = jnp.full((LANES,), -jnp.inf, jnp.float32)
    int_min = jnp.full((LANES,), jnp.int32(-2147483647 - 1))

    def _allreduce(x, op):
        for s in (1, 2, 4, 8):
            x = op(x, x.at[lane ^ s].get(mode="promise_in_bounds"))
        return x

    bufs, obs = (buf0, buf1), (ob0, ob1)
    sis, sos = (si0, si1), (so0, so1)

    pltpu.async_copy(pre_hbm.at[base_row], buf0, si0)
    pltpu.async_copy(pre_hbm.at[base_row + 1], buf1, si1)

    def pair_body(rr, _):
        for kslot in range(2):
            buf, ob = bufs[kslot], obs[kslot]
            si, so = sis[kslot], sos[kslot]
            r = rr * 2 + kslot
            row = base_row + r
            pltpu.make_async_copy(pre_hbm.at[row], buf, si).wait()

            def p1(i, ms):
                b = i * (4 * LANES)
                m0 = jnp.maximum(ms[0], buf[pl.ds(b, LANES)])
                m1 = jnp.maximum(ms[1], buf[pl.ds(b + LANES, LANES)])
                m2 = jnp.maximum(ms[2], buf[pl.ds(b + 2 * LANES, LANES)])
                m3 = jnp.maximum(ms[3], buf[pl.ds(b + 3 * LANES, LANES)])
                return (m0, m1, m2, m3)

            ms = plsc.parallel_loop(0, nv_row // 4, carry=(ninf,) * 4,
                                    unroll=4)(p1)
            m = _allreduce(jnp.minimum(jnp.minimum(ms[0], ms[1]),
                                       jnp.minimum(ms[2], ms[3])),
                           jnp.minimum)

            def pz(i, c):
                b = i * (4 * LANES)
                for u in range(4):
                    cand[pl.ds(b + u * LANES, LANES)] = ninf
                return c

            plsc.parallel_loop(0, CAP // 4, carry=jnp.int32(0),
                               unroll=4)(pz)

            def p2(i, idxc):
                b = i * (4 * LANES)
                out = []
                for u in range(4):
                    v = buf[pl.ds(b + u * LANES, LANES)]
                    ok = v >= m
                    plsc.store_scatter(cand, [idxc[u]], v, mask=ok)
                    out.append(idxc[u] + jnp.where(ok, LANES, 0))
                return tuple(out)

            idx0 = tuple(jnp.int32(q * qwords) + lane for q in range(4))
            idxc = plsc.parallel_loop(0, nv_row // 4, carry=idx0,
                                      unroll=2)(p2)
            hmax = jnp.maximum(jnp.maximum(idxc[0] - 0 * qwords,
                                           idxc[1] - 1 * qwords),
                               jnp.maximum(idxc[2] - 2 * qwords,
                                           idxc[3] - 3 * qwords))
            nv = lax.shift_right_logical(_allreduce(hmax, jnp.maximum)[0],
                                         4)

            def pk(i, c):
                b = i * LANES
                for q in range(4):
                    ki = lax.bitcast_convert_type(
                        cand[pl.ds(q * qwords + b, LANES)], jnp.int32)
                    candk[pl.ds(q * qwords + b, LANES)] = jnp.where(
                        ki < 0, ki ^ 0x7FFFFFFF, ki)
                return c

            plsc.parallel_loop(0, nv, carry=jnp.int32(0), unroll=2)(pk)

            def sb(t, base):
                step = lax.shift_left(jnp.int32(1), jnp.int32(31) - t)
                candidate = base + step

                def cvec(i, acc):
                    b = i * LANES
                    for q in range(4):
                        k = candk[pl.ds(q * qwords + b, LANES)]
                        acc = acc + jnp.where(k >= candidate, 1, 0)
                    return acc

                accv = plsc.parallel_loop(
                    0, nv, carry=jnp.zeros((LANES,), jnp.int32),
                    unroll=2)(cvec)
                total = _allreduce(accv, jnp.add)
                return jnp.where(total >= K, candidate, base)

            base = lax.fori_loop(0, 32, sb, int_min)
            t64 = lax.bitcast_convert_type(
                jnp.where(base < 0, base ^ 0x7FFFFFFF, base), jnp.float32)

            @pl.when(r >= 2)
            def _():
                pltpu.make_async_copy(ob, lat_hbm.at[row - 2], so).wait()

            def p4(i, c):
                b = i * (4 * LANES)
                for u in range(4):
                    v = buf[pl.ds(b + u * LANES, LANES)]
                    ob[pl.ds(b + u * LANES, LANES)] = jnp.where(
                        v >= t64, v, 0.0)
                return c

            plsc.parallel_loop(0, nv_row // 4, carry=jnp.int32(0),
                               unroll=4)(p4)
            pltpu.async_copy(ob, lat_hbm.at[row], so)

            @pl.when(r + 2 < rows_per_w)
            def _():
                pltpu.async_copy(pre_hbm.at[row + 2], buf, si)
        return 0

    lax.fori_loop(0, rows_per_w // 2, pair_body, 0)

    pltpu.make_async_copy(ob0, lat_hbm.at[base_row + rows_per_w - 2],
                          so0).wait()
    pltpu.make_async_copy(ob1, lat_hbm.at[base_row + rows_per_w - 1],
                          so1).wait()


def kernel(x, W_enc, b_enc, W_dec):
    n, d_model = x.shape
    d_sae = W_enc.shape[0]

    rb_e = min(1024, n)
    cb_e = min(2048, d_sae)
    b2 = b_enc.reshape(1, d_sae)
    pre = pl.pallas_call(
        _encode_body,
        grid=(n // rb_e, d_sae // cb_e),
        in_specs=[
            pl.BlockSpec((rb_e, d_model), lambda i, j: (i, 0)),
            pl.BlockSpec((cb_e, d_model), lambda i, j: (j, 0)),
            pl.BlockSpec((1, cb_e), lambda i, j: (0, j)),
        ],
        out_specs=pl.BlockSpec((rb_e, cb_e), lambda i, j: (i, j)),
        out_shape=jax.ShapeDtypeStruct((n, d_sae), jnp.float32),
    )(x, W_enc, b2)

    n_workers = 32
    rows_per_w = n // n_workers
    mesh = plsc.VectorSubcoreMesh(core_axis_name="c", subcore_axis_name="s")
    topk = functools.partial(
        pl.kernel,
        mesh=mesh,
        compiler_params=pltpu.CompilerParams(needs_layout_passes=False),
        out_type=jax.ShapeDtypeStruct((n, d_sae), jnp.float32),
        scratch_types=[
            pltpu.VMEM((d_sae,), jnp.float32),
            pltpu.VMEM((d_sae,), jnp.float32),
            pltpu.VMEM((d_sae,), jnp.float32),
            pltpu.VMEM((d_sae,), jnp.float32),
            pltpu.VMEM((CAP * LANES,), jnp.float32),
            pltpu.VMEM((CAP * LANES,), jnp.int32),
            pltpu.SemaphoreType.DMA,
            pltpu.SemaphoreType.DMA,
            pltpu.SemaphoreType.DMA,
            pltpu.SemaphoreType.DMA,
        ],
    )(functools.partial(_topk_sc_body, n, d_sae, rows_per_w))
    lat = topk(pre)

    rb_d = min(1024, n)
    cb_d = min(2048, d_sae)
    wd_bf = W_dec.astype(jnp.bfloat16)
    out = pl.pallas_call(
        _decode_body,
        grid=(d_sae // cb_d, n // rb_d),
        in_specs=[
            pl.BlockSpec((rb_d, cb_d), lambda j, i: (i, j)),
            pl.BlockSpec((d_model, cb_d), lambda j, i: (0, j)),
        ],
        out_specs=pl.BlockSpec((n, d_model), lambda j, i: (0, 0)),
        out_shape=jax.ShapeDtypeStruct((n, d_model), jnp.float32),
    )(lat, wd_bf)

    return (out, lat)
